# baseline (speedup 1.0000x reference)
import functools

import jax
import jax.numpy as jnp
from jax import lax
from jax.experimental import pallas as pl
from jax.experimental.pallas import tpu as pltpu
from jax.experimental.pallas import tpu_sc as plsc

NC, NS, LANES = 2, 16, 16
NW = NC * NS
CHUNK = 128
ACHUNK = 64


def _sc_mesh():
    return plsc.VectorSubcoreMesh(core_axis_name="c", subcore_axis_name="s")


def _dedup_body(n_nodes, n_edges, ep, src_hbm, dst_hbm, dsteff_hbm, deg_hbm,
                table_hbm, src_v, dst_v, key_v, dsteff_v, eid_v, w_v, ones_v,
                zdeg_v, deg_sh, sem):
    cid = lax.axis_index("c")
    sid = lax.axis_index("s")
    nchunks_t = ep // CHUNK // NS
    rows_t = 640
    nn = n_nodes * n_nodes

    @pl.when(cid == 0)
    def _zero():
        def zb(g, _):
            zdeg_v[pl.ds(g * LANES, LANES)] = jnp.zeros((LANES,), jnp.float32)
            return _
        lax.fori_loop(0, rows_t // LANES, zb, 0)
        for u in range(8):
            ones_v[pl.ds(u * LANES, LANES)] = jnp.ones((LANES,), jnp.float32)
        pltpu.sync_copy(zdeg_v, deg_sh.at[pl.ds(sid * rows_t, rows_t)])

    plsc.subcore_barrier()

    @pl.when(cid == 0)
    def _phase1():
        base_c = sid * nchunks_t
        pltpu.sync_copy(src_hbm.at[pl.ds(base_c, nchunks_t)], src_v)
        pltpu.sync_copy(dst_hbm.at[pl.ds(base_c, nchunks_t)], dst_v)

        def chunk1(j, _):
            ebase = (base_c + j) * CHUNK
            for u in range(CHUNK // LANES):
                s16 = src_v[j, pl.ds(u * LANES, LANES)]
                d16 = dst_v[j, pl.ds(u * LANES, LANES)]
                e16 = ebase + u * LANES + lax.iota(jnp.int32, LANES)
                real = e16 < n_edges
                key = jnp.where(real, s16 * n_nodes + d16, nn + (e16 - n_edges))
                key_v[j, pl.ds(u * LANES, LANES)] = key
                eid_v[j, pl.ds(u * LANES, LANES)] = e16
            pltpu.async_copy(eid_v.at[j], table_hbm.at[key_v.at[j]], sem)
            return _
        lax.fori_loop(0, nchunks_t, chunk1, 0)

        def drain1(j, _):
            pltpu.make_async_copy(eid_v.at[j], table_hbm.at[key_v.at[j]],
                                  sem).wait()
            return _
        lax.fori_loop(0, nchunks_t, drain1, 0)

    plsc.subcore_barrier()

    @pl.when(cid == 0)
    def _phase2():
        base_c = sid * nchunks_t

        def fire2(j, _):
            pltpu.async_copy(table_hbm.at[key_v.at[j]], w_v.at[j], sem)
            return _
        lax.fori_loop(0, nchunks_t, fire2, 0)

        def drain2(j, _):
            pltpu.make_async_copy(table_hbm.at[key_v.at[j]], w_v.at[j],
                                  sem).wait()
            return _
        lax.fori_loop(0, nchunks_t, drain2, 0)

        def chunk2(j, _):
            ebase = (base_c + j) * CHUNK
            for u in range(CHUNK // LANES):
                d16 = dst_v[j, pl.ds(u * LANES, LANES)]
                w16 = w_v[j, pl.ds(u * LANES, LANES)]
                e16 = ebase + u * LANES + lax.iota(jnp.int32, LANES)
                keep = (w16 == e16) & (e16 < n_edges)
                dsteff_v[j, pl.ds(u * LANES, LANES)] = jnp.where(
                    keep, d16, n_nodes)
            pltpu.async_copy(ones_v, deg_sh.at[dsteff_v.at[j]], sem, add=True)
            return _
        lax.fori_loop(0, nchunks_t, chunk2, 0)

        def drain3(j, _):
            pltpu.make_async_copy(ones_v, deg_sh.at[dsteff_v.at[j]],
                                  sem).wait()
            return _
        lax.fori_loop(0, nchunks_t, drain3, 0)
        pltpu.sync_copy(dsteff_v, dsteff_hbm.at[pl.ds(base_c, nchunks_t)])

    plsc.subcore_barrier()

    @pl.when(cid == 0)
    def _wdeg():
        pltpu.sync_copy(deg_sh.at[pl.ds(sid * rows_t, rows_t)],
                        deg_hbm.at[pl.ds(sid * rows_t, rows_t)])


def _make_dedup(n_nodes, n_edges, ep):
    nchunks = ep // CHUNK
    nchunks_t = nchunks // NS
    tbl = n_nodes * n_nodes + (ep - n_edges)
    acc_rows = 640 * NS
    return pl.kernel(
        functools.partial(_dedup_body, n_nodes, n_edges, ep),
        out_type=(
            jax.ShapeDtypeStruct((nchunks, CHUNK), jnp.int32),
            jax.ShapeDtypeStruct((acc_rows,), jnp.float32),
        ),
        mesh=_sc_mesh(),
        scratch_types=[
            pltpu.HBM((tbl,), jnp.int32),
            pltpu.VMEM((nchunks_t, CHUNK), jnp.int32),
            pltpu.VMEM((nchunks_t, CHUNK), jnp.int32),
            pltpu.VMEM((nchunks_t, CHUNK), jnp.int32),
            pltpu.VMEM((nchunks_t, CHUNK), jnp.int32),
            pltpu.VMEM((nchunks_t, CHUNK), jnp.int32),
            pltpu.VMEM((nchunks_t, CHUNK), jnp.int32),
            pltpu.VMEM((CHUNK,), jnp.float32),
            pltpu.VMEM((640,), jnp.float32),
            pltpu.VMEM_SHARED((acc_rows,), jnp.float32),
            pltpu.SemaphoreType.DMA,
        ],
    )


NBUF = 3


def _agg_body(n_nodes, d, ep, h_hbm, src_hbm, dst_hbm, part_hbm,
              src_v, dst_v, r0, r1, r2, acc_sh, s0, s1, s2):
    rows = (r0, r1, r2)
    gsems = (s0, s1, s2)
    zrow_v = r0
    cid = lax.axis_index("c")
    sid = lax.axis_index("s")
    nchunks_t = ep // ACHUNK // NW
    rows_t = 640

    def zb(r, _):
        for u in range(d // LANES):
            zrow_v[r, pl.ds(u * LANES, LANES)] = jnp.zeros((LANES,),
                                                           jnp.float32)
        return _
    lax.fori_loop(0, ACHUNK, zb, 0)
    for k in range(rows_t // ACHUNK):
        pltpu.sync_copy(zrow_v,
                        acc_sh.at[pl.ds(sid * rows_t + k * ACHUNK, ACHUNK)])
    plsc.subcore_barrier()

    base_c = (cid * NS + sid) * nchunks_t
    pltpu.sync_copy(src_hbm.at[pl.ds(base_c, nchunks_t)], src_v)
    pltpu.sync_copy(dst_hbm.at[pl.ds(base_c, nchunks_t)], dst_v)

    for b in range(NBUF):
        pltpu.async_copy(h_hbm.at[src_v.at[b]], rows[b], gsems[b])

    def chunk(j, _):
        for b in range(NBUF):
            jj = j * NBUF + b
            pltpu.make_async_copy(h_hbm.at[src_v.at[jj]], rows[b],
                                  gsems[b]).wait()
            pltpu.sync_copy(rows[b], acc_sh.at[dst_v.at[jj]], add=True)
            pltpu.async_copy(h_hbm.at[src_v.at[jj + NBUF]], rows[b],
                             gsems[b])
        return _
    lax.fori_loop(0, nchunks_t // NBUF - 1, chunk, 0)
    for b in range(NBUF):
        jj = nchunks_t - NBUF + b
        pltpu.make_async_copy(h_hbm.at[src_v.at[jj]], rows[b],
                              gsems[b]).wait()
        pltpu.sync_copy(rows[b], acc_sh.at[dst_v.at[jj]], add=True)

    plsc.subcore_barrier()
    pltpu.sync_copy(acc_sh.at[pl.ds(sid * rows_t, rows_t)],
                    part_hbm.at[cid].at[pl.ds(sid * rows_t, rows_t)])


def _make_agg(n_nodes, d, ep):
    nchunks = ep // ACHUNK
    acc_rows = 640 * NS
    return pl.kernel(
        functools.partial(_agg_body, n_nodes, d, ep),
        out_type=jax.ShapeDtypeStruct((NC, acc_rows, d), jnp.float32),
        mesh=_sc_mesh(),
        scratch_types=[
            pltpu.VMEM((nchunks // NW, ACHUNK), jnp.int32),
            pltpu.VMEM((nchunks // NW, ACHUNK), jnp.int32),
        ] + [pltpu.VMEM((ACHUNK, d), jnp.float32)] * NBUF + [
            pltpu.VMEM_SHARED((acc_rows, d), jnp.float32),
        ] + [pltpu.SemaphoreType.DMA] * NBUF,
    )


def _h0_body(x_ref, u_ref, b_ref, o_ref):
    o_ref[...] = jnp.dot(x_ref[...], u_ref[...],
                         preferred_element_type=jnp.float32) + b_ref[...]


def _layer_body(p_ref, u_ref, deg_ref, o_ref):
    agg = p_ref[0] + p_ref[1]
    z = lax.dot_general(agg, u_ref[...], (((1,), (1,)), ((), ())),
                        preferred_element_type=jnp.float32)
    o_ref[...] = jnp.maximum(z, 0.0) / deg_ref[...]


def _h0_call(x, u0, b0, blk):
    n, d = x.shape
    return pl.pallas_call(
        _h0_body,
        grid=(n // blk,),
        in_specs=[
            pl.BlockSpec((blk, d), lambda i: (i, 0)),
            pl.BlockSpec((d, d), lambda i: (0, 0)),
            pl.BlockSpec((1, d), lambda i: (0, 0)),
        ],
        out_specs=pl.BlockSpec((blk, d), lambda i: (i, 0)),
        out_shape=jax.ShapeDtypeStruct((n, d), jnp.float32),
    )(x, u0, b0.reshape(1, d))


def _layer_call(part, u, deg2d, n, blk):
    d = part.shape[2]
    return pl.pallas_call(
        _layer_body,
        grid=(n // blk,),
        in_specs=[
            pl.BlockSpec((NC, blk, d), lambda i: (0, i, 0)),
            pl.BlockSpec((d, d), lambda i: (0, 0)),
            pl.BlockSpec((blk, 1), lambda i: (i, 0)),
        ],
        out_specs=pl.BlockSpec((blk, d), lambda i: (i, 0)),
        out_shape=jax.ShapeDtypeStruct((n, d), jnp.float32),
    )(part, u, deg2d)


def kernel(x, edge_index, U0, b0, U1, U2):
    n, d = x.shape
    e = edge_index.shape[1]
    ep = ((e + CHUNK * NW - 1) // (CHUNK * NW)) * (CHUNK * NW)
    nchunks = ep // CHUNK

    src = jnp.pad(edge_index[0], (0, ep - e)).reshape(nchunks, CHUNK)
    dst = jnp.pad(edge_index[1], (0, ep - e)).reshape(nchunks, CHUNK)

    dst_eff, deg = _make_dedup(n, e, ep)(src, dst)
    deg2d = deg[:n].reshape(n, 1)

    src_a = src.reshape(ep // ACHUNK, ACHUNK)
    dst_a = dst_eff.reshape(ep // ACHUNK, ACHUNK)
    h = _h0_call(x, U0, b0, 1000)
    agg_fn = _make_agg(n, d, ep)
    for u in (U1, U2):
        part = agg_fn(h, src_a, dst_a)
        h = _layer_call(part, u, deg2d, n, 1000)
    return h


# speedup vs baseline: 1.0453x; 1.0453x over previous
import functools

import jax
import jax.numpy as jnp
from jax import lax
from jax.experimental import pallas as pl
from jax.experimental.pallas import tpu as pltpu
from jax.experimental.pallas import tpu_sc as plsc

NC, NS, LANES = 2, 16, 16
NW = NC * NS
CHUNK = 128
ACHUNK = 64


def _sc_mesh():
    return plsc.VectorSubcoreMesh(core_axis_name="c", subcore_axis_name="s")


def _dedup_body(n_nodes, n_edges, ep, src_hbm, dst_hbm, packed_hbm, deg_hbm,
                table_hbm, src_v, dst_v, key_v, dsteff_v, pk_v, eid_v, w_v,
                ones_v, zdeg_v, deg_sh, sem):
    cid = lax.axis_index("c")
    sid = lax.axis_index("s")
    nchunks_t = ep // CHUNK // NS
    rows_t = 640
    nn = n_nodes * n_nodes

    @pl.when(cid == 0)
    def _zero():
        def zb(g, _):
            zdeg_v[pl.ds(g * LANES, LANES)] = jnp.zeros((LANES,), jnp.float32)
            return _
        lax.fori_loop(0, rows_t // LANES, zb, 0)
        for u in range(8):
            ones_v[pl.ds(u * LANES, LANES)] = jnp.ones((LANES,), jnp.float32)
        pltpu.sync_copy(zdeg_v, deg_sh.at[pl.ds(sid * rows_t, rows_t)])

    plsc.subcore_barrier()

    @pl.when(cid == 0)
    def _phase1():
        base_c = sid * nchunks_t
        pltpu.sync_copy(src_hbm.at[pl.ds(base_c, nchunks_t)], src_v)
        pltpu.sync_copy(dst_hbm.at[pl.ds(base_c, nchunks_t)], dst_v)

        def chunk1(j, _):
            ebase = (base_c + j) * CHUNK
            for u in range(CHUNK // LANES):
                s16 = src_v[j, pl.ds(u * LANES, LANES)]
                d16 = dst_v[j, pl.ds(u * LANES, LANES)]
                e16 = ebase + u * LANES + lax.iota(jnp.int32, LANES)
                real = e16 < n_edges
                key = jnp.where(real, s16 * n_nodes + d16, nn + (e16 - n_edges))
                key_v[j, pl.ds(u * LANES, LANES)] = key
                eid_v[j, pl.ds(u * LANES, LANES)] = e16
            pltpu.async_copy(eid_v.at[j], table_hbm.at[key_v.at[j]], sem)
            return _
        lax.fori_loop(0, nchunks_t, chunk1, 0)

        def drain1(j, _):
            pltpu.make_async_copy(eid_v.at[j], table_hbm.at[key_v.at[j]],
                                  sem).wait()
            return _
        lax.fori_loop(0, nchunks_t, drain1, 0)

    plsc.subcore_barrier()

    @pl.when(cid == 0)
    def _phase2():
        base_c = sid * nchunks_t

        def fire2(j, _):
            pltpu.async_copy(table_hbm.at[key_v.at[j]], w_v.at[j], sem)
            return _
        lax.fori_loop(0, nchunks_t, fire2, 0)

        def drain2(j, _):
            pltpu.make_async_copy(table_hbm.at[key_v.at[j]], w_v.at[j],
                                  sem).wait()
            return _
        lax.fori_loop(0, nchunks_t, drain2, 0)

        def chunk2(j, _):
            ebase = (base_c + j) * CHUNK
            for u in range(CHUNK // LANES):
                s16 = src_v[j, pl.ds(u * LANES, LANES)]
                d16 = dst_v[j, pl.ds(u * LANES, LANES)]
                w16 = w_v[j, pl.ds(u * LANES, LANES)]
                e16 = ebase + u * LANES + lax.iota(jnp.int32, LANES)
                keep = (w16 == e16) & (e16 < n_edges)
                de16 = jnp.where(keep, d16, n_nodes)
                dsteff_v[j, pl.ds(u * LANES, LANES)] = de16
                pk_v[j, pl.ds(u * LANES, LANES)] = s16 * 16384 + de16
            pltpu.async_copy(ones_v, deg_sh.at[dsteff_v.at[j]], sem, add=True)
            return _
        lax.fori_loop(0, nchunks_t, chunk2, 0)

        def drain3(j, _):
            pltpu.make_async_copy(ones_v, deg_sh.at[dsteff_v.at[j]],
                                  sem).wait()
            return _
        lax.fori_loop(0, nchunks_t, drain3, 0)
        pltpu.sync_copy(pk_v, packed_hbm.at[pl.ds(base_c, nchunks_t)])

    plsc.subcore_barrier()

    @pl.when(cid == 0)
    def _wdeg():
        pltpu.sync_copy(deg_sh.at[pl.ds(sid * rows_t, rows_t)],
                        deg_hbm.at[pl.ds(sid * rows_t, rows_t)])


def _make_dedup(n_nodes, n_edges, ep):
    nchunks = ep // CHUNK
    nchunks_t = nchunks // NS
    tbl = n_nodes * n_nodes + (ep - n_edges)
    acc_rows = 640 * NS
    return pl.kernel(
        functools.partial(_dedup_body, n_nodes, n_edges, ep),
        out_type=(
            jax.ShapeDtypeStruct((nchunks, CHUNK), jnp.int32),
            jax.ShapeDtypeStruct((acc_rows,), jnp.float32),
        ),
        mesh=_sc_mesh(),
        scratch_types=[
            pltpu.HBM((tbl,), jnp.int32),
            pltpu.VMEM((nchunks_t, CHUNK), jnp.int32),
            pltpu.VMEM((nchunks_t, CHUNK), jnp.int32),
            pltpu.VMEM((nchunks_t, CHUNK), jnp.int32),
            pltpu.VMEM((nchunks_t, CHUNK), jnp.int32),
            pltpu.VMEM((nchunks_t, CHUNK), jnp.int32),
            pltpu.VMEM((nchunks_t, CHUNK), jnp.int32),
            pltpu.VMEM((nchunks_t, CHUNK), jnp.int32),
            pltpu.VMEM((CHUNK,), jnp.float32),
            pltpu.VMEM((640,), jnp.float32),
            pltpu.VMEM_SHARED((acc_rows,), jnp.float32),
            pltpu.SemaphoreType.DMA,
        ],
    )


NBUF = 4


def _agg_body(n_nodes, d, ep, h_hbm, pk_hbm, part_hbm,
              pk_v, i0, i1, i2, i3, o0, o1, o2, o3, r0, r1, r2, r3, acc_sh,
              s0, s1, s2, s3):
    rows = (r0, r1, r2, r3)
    srcb = (i0, i1, i2, i3)
    dstb = (o0, o1, o2, o3)
    gsems = (s0, s1, s2, s3)
    zrow_v = r0
    cid = lax.axis_index("c")
    sid = lax.axis_index("s")
    nchunks_t = ep // ACHUNK // NW
    nrows_t = ep // CHUNK // NW
    rows_t = 640

    def zb(r, _):
        for u in range(d // LANES):
            zrow_v[r, pl.ds(u * LANES, LANES)] = jnp.zeros((LANES,),
                                                           jnp.float32)
        return _
    lax.fori_loop(0, ACHUNK, zb, 0)
    for k in range(rows_t // ACHUNK):
        pltpu.sync_copy(zrow_v,
                        acc_sh.at[pl.ds(sid * rows_t + k * ACHUNK, ACHUNK)])
    plsc.subcore_barrier()

    base_r = (cid * NS + sid) * nrows_t
    pltpu.sync_copy(pk_hbm.at[pl.ds(base_r, nrows_t)], pk_v)

    def unpack(jj, b):
        r, half = jj // 2, (jj % 2) * ACHUNK
        for u in range(ACHUNK // LANES):
            p16 = pk_v[r, pl.ds(half + u * LANES, LANES)]
            srcb[b][pl.ds(u * LANES, LANES)] = lax.shift_right_logical(
                p16, 14)
            dstb[b][pl.ds(u * LANES, LANES)] = p16 & 16383
        pltpu.async_copy(h_hbm.at[srcb[b]], rows[b], gsems[b])

    for b in range(NBUF):
        unpack(b, b)

    def chunk(j, _):
        for b in range(NBUF):
            jj = j * NBUF + b
            pltpu.make_async_copy(h_hbm.at[srcb[b]], rows[b],
                                  gsems[b]).wait()
            pltpu.sync_copy(rows[b], acc_sh.at[dstb[b]], add=True)
            unpack(jj + NBUF, b)
        return _
    lax.fori_loop(0, nchunks_t // NBUF - 1, chunk, 0)
    for b in range(NBUF):
        pltpu.make_async_copy(h_hbm.at[srcb[b]], rows[b], gsems[b]).wait()
        pltpu.sync_copy(rows[b], acc_sh.at[dstb[b]], add=True)

    plsc.subcore_barrier()
    pltpu.sync_copy(acc_sh.at[pl.ds(sid * rows_t, rows_t)],
                    part_hbm.at[cid].at[pl.ds(sid * rows_t, rows_t)])


def _make_agg(n_nodes, d, ep):
    acc_rows = 640 * NS
    return pl.kernel(
        functools.partial(_agg_body, n_nodes, d, ep),
        out_type=jax.ShapeDtypeStruct((NC, acc_rows, d), jnp.float32),
        mesh=_sc_mesh(),
        scratch_types=[
            pltpu.VMEM((ep // CHUNK // NW, CHUNK), jnp.int32),
        ] + [pltpu.VMEM((ACHUNK,), jnp.int32)] * (2 * NBUF) + [
            pltpu.VMEM((ACHUNK, d), jnp.float32)] * NBUF + [
            pltpu.VMEM_SHARED((acc_rows, d), jnp.float32),
        ] + [pltpu.SemaphoreType.DMA] * NBUF,
    )


def _h0_body(x_ref, u_ref, b_ref, o_ref):
    o_ref[...] = jnp.dot(x_ref[...], u_ref[...],
                         preferred_element_type=jnp.float32) + b_ref[...]


def _layer_body(p_ref, u_ref, deg_ref, o_ref):
    agg = p_ref[0] + p_ref[1]
    z = lax.dot_general(agg, u_ref[...], (((1,), (1,)), ((), ())),
                        preferred_element_type=jnp.float32)
    o_ref[...] = jnp.maximum(z, 0.0) / deg_ref[...]


def _h0_call(x, u0, b0, blk):
    n, d = x.shape
    return pl.pallas_call(
        _h0_body,
        grid=(n // blk,),
        in_specs=[
            pl.BlockSpec((blk, d), lambda i: (i, 0)),
            pl.BlockSpec((d, d), lambda i: (0, 0)),
            pl.BlockSpec((1, d), lambda i: (0, 0)),
        ],
        out_specs=pl.BlockSpec((blk, d), lambda i: (i, 0)),
        out_shape=jax.ShapeDtypeStruct((n, d), jnp.float32),
    )(x, u0, b0.reshape(1, d))


def _layer_call(part, u, deg2d, n, blk):
    d = part.shape[2]
    return pl.pallas_call(
        _layer_body,
        grid=(n // blk,),
        in_specs=[
            pl.BlockSpec((NC, blk, d), lambda i: (0, i, 0)),
            pl.BlockSpec((d, d), lambda i: (0, 0)),
            pl.BlockSpec((blk, 1), lambda i: (i, 0)),
        ],
        out_specs=pl.BlockSpec((blk, d), lambda i: (i, 0)),
        out_shape=jax.ShapeDtypeStruct((n, d), jnp.float32),
    )(part, u, deg2d)


def kernel(x, edge_index, U0, b0, U1, U2):
    n, d = x.shape
    e = edge_index.shape[1]
    ep = ((e + CHUNK * NW - 1) // (CHUNK * NW)) * (CHUNK * NW)
    nchunks = ep // CHUNK

    src = jnp.pad(edge_index[0], (0, ep - e)).reshape(nchunks, CHUNK)
    dst = jnp.pad(edge_index[1], (0, ep - e)).reshape(nchunks, CHUNK)

    packed, deg = _make_dedup(n, e, ep)(src, dst)
    deg2d = deg[:n].reshape(n, 1)

    h = _h0_call(x, U0, b0, 1000)
    agg_fn = _make_agg(n, d, ep)
    for u in (U1, U2):
        part = agg_fn(h, packed)
        h = _layer_call(part, u, deg2d, n, 1000)
    return h
